# baseline (speedup 1.0000x reference)
import jax
import jax.numpy as jnp
from jax import lax
from jax.experimental import pallas as pl
from jax.experimental.pallas import tpu as pltpu
from jax.experimental.pallas import tpu_sc as plsc

CAP = 1000000
N_WRITE = 16384
BATCH = 4096
DUMP = 64
NC = 2
NS = 16
L = 16
WPT = N_WRITE // NS
SPT = BATCH // (NC * NS)
MAP_N = 1000064

_MESH = dict(core_axis_name="c", subcore_axis_name="s",
             num_cores=NC, num_subcores=NS)
_PARAMS = dict(use_tc_tiling_on_sc=False, needs_layout_passes=False)


def _join_body(widx, sidx, out_g,
               map_sh, wt, ct, wf, st, gv, gb, fs, fp, zi):
    c = lax.axis_index("c")
    s = lax.axis_index("s")
    w = s * NC + c
    iota = lax.iota(jnp.int32, L)

    for i in range(8):
        pltpu.sync_copy(widx.at[pl.ds(s * WPT + i * 128, 128)], wt.at[i])
    pltpu.sync_copy(widx, wf)
    pltpu.sync_copy(sidx.at[pl.ds(w * SPT, SPT)], st)

    for i in range(8):
        for t in range(8):
            ct[i, pl.ds(t * L, L)] = iota + (65536 + s * WPT + i * 128 + t * L)
    for t in range(8):
        zi[pl.ds(t * L, L)] = jnp.int32(0) * iota

    for i in range(8):
        pltpu.sync_copy(zi, map_sh.at[wt.at[i]])
    pltpu.sync_copy(zi, map_sh.at[st])
    plsc.subcore_barrier()

    for i in range(8):
        pltpu.sync_copy(ct.at[i], map_sh.at[wt.at[i]], add=True)
    plsc.subcore_barrier()

    pltpu.sync_copy(map_sh.at[st], gv)

    o = jnp.int32(0)
    for i in range(8):
        v = gv[pl.ds(i * L, L)]
        hi = v >> 16
        lo = v & 65535
        sv = st[pl.ds(i * L, L)]
        pos = iota + i * L
        gb[pl.ds(i * L, L)] = jnp.where(hi == 1, lo + 1, 0)
        need = hi >= 2
        ni = need.astype(jnp.int32)
        csum = jnp.cumsum(ni)
        dst = o + csum - ni
        plsc.store_scatter(fs, [dst], sv, mask=need)
        plsc.store_scatter(fp, [dst], pos, mask=need)
        o = o + jnp.sum(ni)

    def fb(e, carry):
        sv = fs[pl.ds(e, L)]
        s_val = jnp.sum(jnp.where(iota == 0, sv, 0))
        pv = fp[pl.ds(e, L)]
        p_val = jnp.sum(jnp.where(iota == 0, pv, 0))

        def scan(k, best):
            wv = wf[pl.ds(k * L, L)]
            jv = iota + (k * L + 1)
            return jnp.maximum(best, jnp.where(wv == s_val, jv, 0))

        best_v = lax.fori_loop(0, N_WRITE // L, scan,
                               jnp.zeros((L,), jnp.int32))
        best = jnp.max(best_v)
        zv = jnp.zeros((L,), jnp.int32)
        plsc.store_scatter(gb, [p_val + zv], best + zv, mask=iota == 0)
        return carry

    lax.fori_loop(0, o, fb, jnp.int32(0))
    pltpu.sync_copy(gb, out_g.at[pl.ds(w * SPT, SPT)])


def _mat_body(g_in, st_in, ns_in, ar_in, out_s, out_n, out_ar,
              gb, gs, ga, stv, nsv, arv, oS, oN, oA):
    c = lax.axis_index("c")
    s = lax.axis_index("s")
    w = s * NC + c
    iota = lax.iota(jnp.int32, L)

    pltpu.sync_copy(g_in.at[pl.ds(w * SPT, SPT)], gb)

    for i in range(8):
        g = gb[pl.ds(i * L, L)]
        pos = iota + i * L
        wr = g > 0
        j = g - 1
        gs[pl.ds(i * L, L)] = jnp.where(wr, j >> 2, pos & 63)
        ga[pl.ds(i * L, L)] = jnp.where(wr, j >> 3, pos & 63)
    pltpu.sync_copy(st_in.at[gs], stv)
    pltpu.sync_copy(ns_in.at[gs], nsv)
    pltpu.sync_copy(ar_in.at[ga], arv)

    for i in range(8):
        rows = iota + i * L
        g = gb[pl.ds(i * L, L)]
        wr = g > 0
        j = g - 1
        offs = (j & 3) * 32
        offa = (j & 7) * 16

        def col_s(cc, carry):
            cv = iota * 0 + cc
            vs = plsc.load_gather(stv, [rows, offs + cc])
            plsc.store_scatter(oS, [rows, cv], jnp.where(wr, vs, 0.0))
            vn = plsc.load_gather(nsv, [rows, offs + cc])
            plsc.store_scatter(oN, [rows, cv], jnp.where(wr, vn, 0.0))
            return carry

        lax.fori_loop(0, 32, col_s, jnp.int32(0))

        def col_a(cc, carry):
            cv = iota * 0 + cc
            va = plsc.load_gather(arv, [rows, offa + cc])
            plsc.store_scatter(oA, [rows, cv], jnp.where(wr, va, 0.0))
            return carry

        lax.fori_loop(0, 16, col_a, jnp.int32(0))

    pltpu.sync_copy(oS, out_s.at[pl.ds(w * SPT, SPT)])
    pltpu.sync_copy(oN, out_n.at[pl.ds(w * SPT, SPT)])
    pltpu.sync_copy(oA, out_ar.at[pl.ds(w * SPT, SPT)])


@jax.jit
def _sc_call(widx, sidx, st_in, ns_in, ar_in):
    g = pl.kernel(
        _join_body,
        out_type=jax.ShapeDtypeStruct((BATCH,), jnp.int32),
        mesh=plsc.VectorSubcoreMesh(**_MESH),
        compiler_params=pltpu.CompilerParams(**_PARAMS),
        scratch_types=[
            pltpu.VMEM_SHARED((MAP_N,), jnp.int32),
            pltpu.VMEM((8, 128), jnp.int32),
            pltpu.VMEM((8, 128), jnp.int32),
            pltpu.VMEM((N_WRITE,), jnp.int32),
            pltpu.VMEM((SPT,), jnp.int32),
            pltpu.VMEM((SPT,), jnp.int32),
            pltpu.VMEM((SPT,), jnp.int32),
            pltpu.VMEM((SPT + L,), jnp.int32),
            pltpu.VMEM((SPT + L,), jnp.int32),
            pltpu.VMEM((SPT,), jnp.int32),
        ],
    )(widx, sidx)
    return pl.kernel(
        _mat_body,
        out_type=(
            jax.ShapeDtypeStruct((BATCH, 32), jnp.float32),
            jax.ShapeDtypeStruct((BATCH, 32), jnp.float32),
            jax.ShapeDtypeStruct((BATCH, 16), jnp.float32),
        ),
        mesh=plsc.VectorSubcoreMesh(**_MESH),
        compiler_params=pltpu.CompilerParams(
            use_tc_tiling_on_sc=True, needs_layout_passes=False),
        scratch_types=[
            pltpu.VMEM((SPT,), jnp.int32),
            pltpu.VMEM((SPT,), jnp.int32),
            pltpu.VMEM((SPT,), jnp.int32),
            pltpu.VMEM((SPT, 128), jnp.float32),
            pltpu.VMEM((SPT, 128), jnp.float32),
            pltpu.VMEM((SPT, 128), jnp.float32),
            pltpu.VMEM((SPT, 32), jnp.float32),
            pltpu.VMEM((SPT, 32), jnp.float32),
            pltpu.VMEM((SPT, 16), jnp.float32),
        ],
    )(g, st_in, ns_in, ar_in)


def kernel(state_buffer, action_buffer, reward_buffer, next_state_buffer,
           new_states, new_actions, new_rewards, new_next_states,
           write_idx, sample_idx):
    widx = write_idx.astype(jnp.int32)
    sidx = sample_idx.astype(jnp.int32)
    ar = jnp.concatenate(
        [new_actions, new_rewards, jnp.zeros((N_WRITE, 7), jnp.float32)],
        axis=1)
    out_s, out_n, out_ar = _sc_call(
        widx, sidx,
        new_states.reshape(BATCH, 128),
        new_next_states.reshape(BATCH, 128),
        ar.reshape(N_WRITE // 8, 128))
    return (out_s, out_ar[:, :8], out_ar[:, 8:9], out_n)


# speedup vs baseline: 1.1619x; 1.1619x over previous
import jax
import jax.numpy as jnp
from jax import lax
from jax.experimental import pallas as pl
from jax.experimental.pallas import tpu as pltpu
from jax.experimental.pallas import tpu_sc as plsc

CAP = 1000000
N_WRITE = 16384
BATCH = 4096
DUMP = 64
NC = 2
NS = 16
L = 16
WPT = N_WRITE // NS
SPT = BATCH // (NC * NS)
MAP_N = 1000064

_MESH = dict(core_axis_name="c", subcore_axis_name="s",
             num_cores=NC, num_subcores=NS)
_PARAMS = dict(use_tc_tiling_on_sc=False, needs_layout_passes=False)


def _join_body(widx, sidx, out_g,
               map_sh, wt, ct, wf, st, gv, gb, fs, fp, zi):
    c = lax.axis_index("c")
    s = lax.axis_index("s")
    w = s * NC + c
    iota = lax.iota(jnp.int32, L)

    for i in range(8):
        pltpu.sync_copy(widx.at[pl.ds(s * WPT + i * 128, 128)], wt.at[i])
    pltpu.sync_copy(widx, wf)
    pltpu.sync_copy(sidx.at[pl.ds(w * SPT, SPT)], st)

    for i in range(8):
        for t in range(8):
            ct[i, pl.ds(t * L, L)] = iota + (65536 + s * WPT + i * 128 + t * L)
    for t in range(8):
        zi[pl.ds(t * L, L)] = jnp.int32(0) * iota

    for i in range(8):
        pltpu.sync_copy(zi, map_sh.at[wt.at[i]])
    pltpu.sync_copy(zi, map_sh.at[st])
    plsc.subcore_barrier()

    for i in range(8):
        pltpu.sync_copy(ct.at[i], map_sh.at[wt.at[i]], add=True)
    plsc.subcore_barrier()

    pltpu.sync_copy(map_sh.at[st], gv)

    o = jnp.int32(0)
    for i in range(8):
        v = gv[pl.ds(i * L, L)]
        hi = v >> 16
        lo = v & 65535
        sv = st[pl.ds(i * L, L)]
        pos = iota + i * L
        gb[pl.ds(i * L, L)] = jnp.where(hi == 1, lo + 1, 0)
        need = hi >= 2
        ni = need.astype(jnp.int32)
        csum = jnp.cumsum(ni)
        dst = o + csum - ni
        plsc.store_scatter(fs, [dst], sv, mask=need)
        plsc.store_scatter(fp, [dst], pos, mask=need)
        o = o + jnp.sum(ni)

    def fb(e, carry):
        sv = fs[pl.ds(e, L)]
        s_val = jnp.sum(jnp.where(iota == 0, sv, 0))
        pv = fp[pl.ds(e, L)]
        p_val = jnp.sum(jnp.where(iota == 0, pv, 0))

        def scan(k, best):
            wv = wf[pl.ds(k * L, L)]
            jv = iota + (k * L + 1)
            return jnp.maximum(best, jnp.where(wv == s_val, jv, 0))

        best_v = lax.fori_loop(0, N_WRITE // L, scan,
                               jnp.zeros((L,), jnp.int32))
        best = jnp.max(best_v)
        zv = jnp.zeros((L,), jnp.int32)
        plsc.store_scatter(gb, [p_val + zv], best + zv, mask=iota == 0)
        return carry

    lax.fori_loop(0, o, fb, jnp.int32(0))
    pltpu.sync_copy(gb, out_g.at[pl.ds(w * SPT, SPT)])


def _mat_body(g_in, st_in, ns_in, ar_in, out_s, out_n, out_ar,
              gb, gi, op, stv, nsv, arv, zA, zB, sem):
    c = lax.axis_index("c")
    s = lax.axis_index("s")
    w = s * NC + c
    iota = lax.iota(jnp.int32, L)
    zf = jnp.zeros((L,), jnp.float32)

    pltpu.sync_copy(g_in.at[pl.ds(w * SPT, SPT)], gb)

    def zrow(r, carry):
        zA[r, pl.ds(0, L)] = zf
        zA[r, pl.ds(L, L)] = zf
        zB[r, pl.ds(0, L)] = zf
        return carry

    lax.fori_loop(0, SPT, zrow, jnp.int32(0))

    for i in range(8):
        g = gb[pl.ds(i * L, L)]
        pos = iota + i * L
        gi[pl.ds(i * L, L)] = jnp.where(g > 0, g - 1, pos & (DUMP - 1))
        op[pl.ds(i * L, L)] = jnp.where(g > 0, w * SPT + pos,
                                        BATCH + (pos & (DUMP - 1)))

    d = [
        pltpu.async_copy(zA, out_s.at[pl.ds(w * SPT, SPT)], sem),
        pltpu.async_copy(zA, out_n.at[pl.ds(w * SPT, SPT)], sem),
        pltpu.async_copy(zB, out_ar.at[pl.ds(w * SPT, SPT)], sem),
        pltpu.async_copy(st_in.at[gi], stv, sem),
        pltpu.async_copy(ns_in.at[gi], nsv, sem),
        pltpu.async_copy(ar_in.at[gi], arv, sem),
    ]
    for a in d:
        a.wait()
    d = [
        pltpu.async_copy(stv, out_s.at[op], sem),
        pltpu.async_copy(nsv, out_n.at[op], sem),
        pltpu.async_copy(arv, out_ar.at[op], sem),
    ]
    for a in d:
        a.wait()


@jax.jit
def _sc_call(widx, sidx, st_in, ns_in, ar_in):
    g = pl.kernel(
        _join_body,
        out_type=jax.ShapeDtypeStruct((BATCH,), jnp.int32),
        mesh=plsc.VectorSubcoreMesh(**_MESH),
        compiler_params=pltpu.CompilerParams(**_PARAMS),
        scratch_types=[
            pltpu.VMEM_SHARED((MAP_N,), jnp.int32),
            pltpu.VMEM((8, 128), jnp.int32),
            pltpu.VMEM((8, 128), jnp.int32),
            pltpu.VMEM((N_WRITE,), jnp.int32),
            pltpu.VMEM((SPT,), jnp.int32),
            pltpu.VMEM((SPT,), jnp.int32),
            pltpu.VMEM((SPT,), jnp.int32),
            pltpu.VMEM((SPT + L,), jnp.int32),
            pltpu.VMEM((SPT + L,), jnp.int32),
            pltpu.VMEM((SPT,), jnp.int32),
        ],
    )(widx, sidx)
    return pl.kernel(
        _mat_body,
        out_type=(
            jax.ShapeDtypeStruct((BATCH + DUMP, 32), jnp.float32),
            jax.ShapeDtypeStruct((BATCH + DUMP, 32), jnp.float32),
            jax.ShapeDtypeStruct((BATCH + DUMP, 16), jnp.float32),
        ),
        mesh=plsc.VectorSubcoreMesh(**_MESH),
        compiler_params=pltpu.CompilerParams(**_PARAMS),
        scratch_types=[
            pltpu.VMEM((SPT,), jnp.int32),
            pltpu.VMEM((SPT,), jnp.int32),
            pltpu.VMEM((SPT,), jnp.int32),
            pltpu.VMEM((SPT, 32), jnp.float32),
            pltpu.VMEM((SPT, 32), jnp.float32),
            pltpu.VMEM((SPT, 16), jnp.float32),
            pltpu.VMEM((SPT, 32), jnp.float32),
            pltpu.VMEM((SPT, 16), jnp.float32),
            pltpu.SemaphoreType.DMA,
        ],
    )(g, st_in, ns_in, ar_in)


def kernel(state_buffer, action_buffer, reward_buffer, next_state_buffer,
           new_states, new_actions, new_rewards, new_next_states,
           write_idx, sample_idx):
    widx = write_idx.astype(jnp.int32)
    sidx = sample_idx.astype(jnp.int32)
    ar = jnp.concatenate(
        [new_actions, new_rewards, jnp.zeros((N_WRITE, 7), jnp.float32)],
        axis=1)
    out_s, out_n, out_ar = _sc_call(widx, sidx, new_states, new_next_states, ar)
    return (out_s[:BATCH], out_ar[:BATCH, :8], out_ar[:BATCH, 8:9],
            out_n[:BATCH])
